# baseline (speedup 1.0000x reference)
import functools

import jax
import jax.numpy as jnp
from jax import lax
from jax.experimental import pallas as pl
from jax.experimental.pallas import tpu as pltpu
from jax.experimental.pallas import tpu_sc as plsc


N_NODES = 10000
D = 128
G_GRAPHS = 128
ROW_BLK = 2048

NC = 2
NS = 16
NW = NC * NS
CHUNK = 64
CHUNKS_PER_S = 320
CPW0 = 160
CPW1 = 160
GROUP = 24
GROUP_LENS = (24, 24, 24, 24, 24, 24, 16)
E_PAD = NS * CHUNKS_PER_S * CHUNK
EDGES_PER_HIST_W = E_PAD // NW
HR = 80
N_ACC = 10240
ROWS_PER_TILE = N_ACC // NS


def _edge_agg_body(g_hbm, src_hbm, dst_hbm, out_hbm,
                   src_idx, dst_idx, buf0, buf1, buf2, acc,
                   sem0, sem1, sem2, ssem0, ssem1, ssem2):
    c = lax.axis_index("c")
    s = lax.axis_index("s")
    w = s * NC + c

    with jax.named_scope("agg_zero"):
        def zero_row(i, carry):
            for f in range(D // 16):
                buf0[i, pl.ds(f * 16, 16)] = jnp.zeros((16,), jnp.float32)
            return carry
        lax.fori_loop(0, CHUNK, zero_row, None)
        base = s * ROWS_PER_TILE
        for k in range(ROWS_PER_TILE // CHUNK):
            pltpu.sync_copy(buf0, acc.at[pl.ds(base + k * CHUNK, CHUNK)])
        rem = ROWS_PER_TILE % CHUNK
        if rem:
            pltpu.sync_copy(buf0.at[pl.ds(0, rem)],
                            acc.at[pl.ds(base + ROWS_PER_TILE - rem, rem)])
        plsc.subcore_barrier()

    bufs = (buf0, buf1, buf2)
    sems = (sem0, sem1, sem2)
    ssems = (ssem0, ssem1, ssem2)

    def start(j, b):
        pltpu.async_copy(g_hbm.at[src_idx.at[j]], bufs[b], sems[b])

    def wait(j, b):
        pltpu.make_async_copy(g_hbm.at[src_idx.at[j]], bufs[b], sems[b]).wait()

    def start_sc(j, b):
        pltpu.async_copy(bufs[b], acc.at[dst_idx.at[j]], ssems[b], add=True)

    def wait_sc(j, b):
        pltpu.make_async_copy(bufs[b], acc.at[dst_idx.at[j]], ssems[b]).wait()

    def run_groups(first_chunk):
        done = 0
        for glen in GROUP_LENS:
            g0 = first_chunk + done
            pltpu.sync_copy(src_hbm.at[pl.ds(g0, glen)],
                            src_idx.at[pl.ds(0, glen)])
            pltpu.sync_copy(dst_hbm.at[pl.ds(g0, glen)],
                            dst_idx.at[pl.ds(0, glen)])
            for j in range(min(3, glen)):
                start(j, j % 3)
            for j in range(glen):
                b = j % 3
                wait(j, b)
                start_sc(j, b)
                jj = j - 2
                if jj >= 0 and jj + 3 < glen:
                    wait_sc(jj, jj % 3)
                    start(jj + 3, jj % 3)
            for jj in range(max(0, glen - 3), glen):
                wait_sc(jj, jj % 3)
            done += glen

    with jax.named_scope("agg_edges"):
        base_chunk = s * CHUNKS_PER_S
        pl.when(c == 0)(lambda: run_groups(base_chunk))
        pl.when(c == 1)(lambda: run_groups(base_chunk + CPW0))
        plsc.subcore_barrier()

    with jax.named_scope("agg_writeout"):
        for k in range(ROWS_PER_TILE // CHUNK):
            b = base + k * CHUNK
            pltpu.sync_copy(acc.at[pl.ds(b, CHUNK)], buf0)
            pltpu.sync_copy(buf0, out_hbm.at[c, pl.ds(b, CHUNK)])
        if rem:
            b = base + ROWS_PER_TILE - rem
            pltpu.sync_copy(acc.at[pl.ds(b, rem)], buf0.at[pl.ds(0, rem)])
            pltpu.sync_copy(buf0.at[pl.ds(0, rem)],
                            out_hbm.at[c, pl.ds(b, rem)])


@functools.partial(
    pl.kernel,
    mesh=plsc.VectorSubcoreMesh(core_axis_name="c", subcore_axis_name="s"),
    out_type=jax.ShapeDtypeStruct((NC, N_ACC, D), jnp.float32),
    scratch_types=[
        pltpu.VMEM((GROUP, CHUNK), jnp.int32),
        pltpu.VMEM((GROUP, CHUNK), jnp.int32),
        pltpu.VMEM((CHUNK, D), jnp.float32),
        pltpu.VMEM((CHUNK, D), jnp.float32),
        pltpu.VMEM((CHUNK, D), jnp.float32),
        pltpu.VMEM_SHARED((N_ACC, D), jnp.float32),
    ] + [pltpu.SemaphoreType.DMA] * 6,
)
def _edge_agg(g_hbm, src_hbm, dst_hbm, out_hbm,
              src_idx, dst_idx, buf0, buf1, buf2, acc,
              sem0, sem1, sem2, ssem0, ssem1, ssem2):
    _edge_agg_body(g_hbm, src_hbm, dst_hbm, out_hbm,
                   src_idx, dst_idx, buf0, buf1, buf2, acc,
                   sem0, sem1, sem2, ssem0, ssem1, ssem2)


DEG_HIST = NW * (E_PAD // NW // NW)
HIST_N = 10240
DC = 128
DCH = 80
DGROUP = 16


def _deg_body(dst_hbm, degp_hbm, dst_idx, ones_v, zb, wb, hist, sem):
    c = lax.axis_index("c")
    s = lax.axis_index("s")
    w = s * NC + c
    zero16 = jnp.zeros((16,), jnp.float32)

    def zrow(i, carry):
        zb[pl.ds(i * 16, 16)] = zero16
        return carry
    lax.fori_loop(0, (HIST_N // NS) // 16, zrow, None)
    pltpu.sync_copy(zb, hist.at[pl.ds(s * (HIST_N // NS), HIST_N // NS)])
    for k in range(DC // 16):
        ones_v[pl.ds(k * 16, 16)] = jnp.ones((16,), jnp.float32)
    plsc.subcore_barrier()

    for grp in range(DCH // DGROUP):
        g0 = w * DCH + grp * DGROUP
        pltpu.sync_copy(dst_hbm.at[pl.ds(g0, DGROUP)], dst_idx)
        for j in range(DGROUP):
            pltpu.async_copy(ones_v, hist.at[dst_idx.at[j]], sem, add=True)
        for j in range(DGROUP):
            pltpu.make_async_copy(ones_v, hist.at[dst_idx.at[j]], sem).wait()
    plsc.subcore_barrier()

    def wout():
        q = HIST_N // 4
        pltpu.sync_copy(hist.at[pl.ds(s * q, q)], wb)
        pltpu.sync_copy(wb, degp_hbm.at[c, pl.ds(s * q, q)])
    pl.when(s < 4)(wout)


@functools.partial(
    pl.kernel,
    mesh=plsc.VectorSubcoreMesh(core_axis_name="c", subcore_axis_name="s"),
    out_type=jax.ShapeDtypeStruct((NC, HIST_N), jnp.float32),
    scratch_types=[
        pltpu.VMEM((DGROUP, DC), jnp.int32),
        pltpu.VMEM((DC,), jnp.float32),
        pltpu.VMEM((HIST_N // NS,), jnp.float32),
        pltpu.VMEM((HIST_N // 4,), jnp.float32),
        pltpu.VMEM_SHARED((HIST_N,), jnp.float32),
        pltpu.SemaphoreType.DMA,
    ],
)
def _deg_hist(dst_hbm, degp_hbm, dst_idx, ones_v, zb, wb, hist, sem):
    _deg_body(dst_hbm, degp_hbm, dst_idx, ones_v, zb, wb, hist, sem)


SEG_ROWS_PER_W = N_ACC // NW
SEG_CHUNK = 80


def _seg_body(parts_hbm, g_hbm, dinv_hbm, b1_hbm, batch_hbm,
              segp_hbm, rootsp_hbm,
              a0b, a1b, gb, dinvv, batchv, b1v, seg, roots, sems):
    c = lax.axis_index("c")
    s = lax.axis_index("s")
    w = s * NC + c
    neg = jnp.full((16,), -jnp.inf, jnp.float32)

    def irow(i, carry):
        seg[pl.ds(i * 16, 16)] = neg
        return carry
    lax.fori_loop(0, (G_GRAPHS + 1) * D // 16, irow, None)

    def irt(i, carry):
        roots[pl.ds(i * 16, 16)] = jnp.full((16,), -1, jnp.int32)
        return carry
    lax.fori_loop(0, G_GRAPHS + 1, irt, None)

    first = w * SEG_ROWS_PER_W
    off0 = jnp.where(w == 0, 0, first - 8)
    pltpu.sync_copy(batch_hbm.at[pl.ds(off0, 16)],
                    batchv.at[0, pl.ds(0, 16)])
    cur0 = jnp.where(w == 0, jnp.int32(-1), batchv[0, pl.ds(0, 16)][7])

    pltpu.sync_copy(b1_hbm, b1v)
    b1r = [b1v[pl.ds(f * 16, 16)] for f in range(D // 16)]

    def flush(b, regs):
        for f in range(D // 16):
            seg[pl.ds(b * D + f * 16, 16)] = regs[f]

    def startch(k, ph):
        base = w * SEG_ROWS_PER_W + k * SEG_CHUNK
        pltpu.async_copy(parts_hbm.at[0, pl.ds(base * D, SEG_CHUNK * D)],
                         a0b.at[ph], sems[0])
        pltpu.async_copy(parts_hbm.at[1, pl.ds(base * D, SEG_CHUNK * D)],
                         a1b.at[ph], sems[1])
        pltpu.async_copy(g_hbm.at[pl.ds(base * D, SEG_CHUNK * D)],
                         gb.at[ph], sems[2])
        pltpu.async_copy(dinv_hbm.at[pl.ds(base, SEG_CHUNK)],
                         dinvv.at[ph, pl.ds(0, SEG_CHUNK)], sems[3])
        pltpu.async_copy(batch_hbm.at[pl.ds(base, SEG_CHUNK)],
                         batchv.at[ph, pl.ds(0, SEG_CHUNK)], sems[4])

    def waitch(k, ph):
        base = w * SEG_ROWS_PER_W + k * SEG_CHUNK
        pltpu.make_async_copy(parts_hbm.at[0, pl.ds(base * D, SEG_CHUNK * D)],
                              a0b.at[ph], sems[0]).wait()
        pltpu.make_async_copy(parts_hbm.at[1, pl.ds(base * D, SEG_CHUNK * D)],
                              a1b.at[ph], sems[1]).wait()
        pltpu.make_async_copy(g_hbm.at[pl.ds(base * D, SEG_CHUNK * D)],
                              gb.at[ph], sems[2]).wait()
        pltpu.make_async_copy(dinv_hbm.at[pl.ds(base, SEG_CHUNK)],
                              dinvv.at[ph, pl.ds(0, SEG_CHUNK)], sems[3]).wait()
        pltpu.make_async_copy(batch_hbm.at[pl.ds(base, SEG_CHUNK)],
                              batchv.at[ph, pl.ds(0, SEG_CHUNK)], sems[4]).wait()

    NCH = SEG_ROWS_PER_W // SEG_CHUNK

    startch(0, 0)
    carry = (cur0,) + tuple(neg for _ in range(D // 16))
    for k in range(NCH):
        ph = k % 2
        waitch(k, ph)
        if k + 1 < NCH:
            startch(k + 1, (k + 1) % 2)
        base = w * SEG_ROWS_PER_W + k * SEG_CHUNK

        def row_body(r, rc, ph=ph, base=base):
            cur_b = rc[0]
            regs = rc[1:]
            b = batchv[ph, pl.ds(r, 16)][0]
            dv = dinvv[ph, pl.ds(r, 16)][0]
            vals = []
            for f in range(D // 16):
                sl = pl.ds(r * D + f * 16, 16)
                v = (a0b[ph, sl] + a1b[ph, sl] + gb[ph, sl]) * dv + b1r[f]
                vals.append(jnp.maximum(v, 0.0))
            is_new = b != cur_b
            pl.when(is_new & (cur_b >= 0))(lambda: flush(cur_b, regs))
            pl.when(is_new)(lambda: roots.__setitem__(
                pl.ds(b * 16, 16), jnp.full((16,), 1, jnp.int32) * (base + r)))
            new_regs = tuple(
                jnp.where(is_new, vals[f], jnp.maximum(regs[f], vals[f]))
                for f in range(D // 16))
            return (b,) + new_regs
        carry = lax.fori_loop(0, SEG_CHUNK, row_body, carry)

    fin = carry
    pl.when(fin[0] >= 0)(lambda: flush(fin[0], fin[1:]))

    pltpu.sync_copy(seg.at[pl.ds(0, G_GRAPHS * D)], segp_hbm.at[w])
    pltpu.sync_copy(roots.at[pl.ds(0, G_GRAPHS * 16)], rootsp_hbm.at[w])


@functools.partial(
    pl.kernel,
    mesh=plsc.VectorSubcoreMesh(core_axis_name="c", subcore_axis_name="s"),
    out_type=(jax.ShapeDtypeStruct((NW, G_GRAPHS * D), jnp.float32),
              jax.ShapeDtypeStruct((NW, G_GRAPHS * 16), jnp.int32)),
    scratch_types=[
        pltpu.VMEM((2, SEG_CHUNK * D), jnp.float32),
        pltpu.VMEM((2, SEG_CHUNK * D), jnp.float32),
        pltpu.VMEM((2, SEG_CHUNK * D), jnp.float32),
        pltpu.VMEM((2, SEG_CHUNK + 16), jnp.float32),
        pltpu.VMEM((2, SEG_CHUNK + 16), jnp.int32),
        pltpu.VMEM((D,), jnp.float32),
        pltpu.VMEM(((G_GRAPHS + 1) * D,), jnp.float32),
        pltpu.VMEM(((G_GRAPHS + 1) * 16,), jnp.int32),
    ] + [pltpu.SemaphoreType.DMA] * 5,
)
def _conv_segmax(parts_hbm, g_hbm, dinv_hbm, b1_hbm, batch_hbm,
                 segp_hbm, rootsp_hbm,
                 a0b, a1b, gb, dinvv, batchv, b1v, seg, roots,
                 sm0, sm1, sm2, sm3, sm4):
    _seg_body(parts_hbm, g_hbm, dinv_hbm, b1_hbm, batch_hbm,
              segp_hbm, rootsp_hbm,
              a0b, a1b, gb, dinvv, batchv, b1v, seg, roots,
              (sm0, sm1, sm2, sm3, sm4))


def _mm_body(x_ref, w_ref, dinv_ref, g_ref):
    h = jnp.dot(x_ref[...], w_ref[...], preferred_element_type=jnp.float32)
    g_ref[...] = h * dinv_ref[...]


def _matmul_scale(x, w, dinv2d):
    n = x.shape[0]
    grid = (n // ROW_BLK,)
    return pl.pallas_call(
        _mm_body,
        grid=grid,
        in_specs=[
            pl.BlockSpec((ROW_BLK, D), lambda i: (i, 0)),
            pl.BlockSpec((D, D), lambda i: (0, 0)),
            pl.BlockSpec((ROW_BLK, 1), lambda i: (i, 0)),
        ],
        out_specs=pl.BlockSpec((ROW_BLK, D), lambda i: (i, 0)),
        out_shape=jax.ShapeDtypeStruct((n, D), jnp.float32),
    )(x, w, dinv2d)


def _heads_body(hp_ref, nx_ref, w2_ref, b2_ref, wn_ref, bn_ref,
                w3a_ref, w3b_ref, b3_ref, out_ref):
    hp = jnp.max(hp_ref[...], axis=0)
    a = jnp.maximum(
        jnp.dot(hp, w2_ref[...], preferred_element_type=jnp.float32)
        + b2_ref[...], 0.0)
    b = jnp.maximum(
        jnp.dot(nx_ref[...], wn_ref[...], preferred_element_type=jnp.float32)
        + bn_ref[...], 0.0)
    z = (jnp.dot(a, w3a_ref[...], preferred_element_type=jnp.float32)
         + jnp.dot(b, w3b_ref[...], preferred_element_type=jnp.float32)
         + b3_ref[...])
    out_ref[...] = jax.nn.sigmoid(z)


def _heads(hp, news_x, lin2_W, lin2_b, linnews_W, linnews_b, lin3_W, lin3_b):
    full = lambda s: pl.BlockSpec(s, lambda: (0,) * len(s))
    return pl.pallas_call(
        _heads_body,
        in_specs=[full((NW, G_GRAPHS, D)), full((G_GRAPHS, D)),
                  full((D, D)), full((1, D)),
                  full((D, D)), full((1, D)),
                  full((D, 1)), full((D, 1)), full((1, 1))],
        out_specs=full((G_GRAPHS, 1)),
        out_shape=jax.ShapeDtypeStruct((G_GRAPHS, 1), jnp.float32),
    )(hp, news_x, lin2_W, lin2_b.reshape(1, D), linnews_W,
      linnews_b.reshape(1, D), lin3_W[:D], lin3_W[D:], lin3_b.reshape(1, 1))


def kernel(x, adj, batch, conv_W0, conv_b0, conv_W1, conv_b1,
           linnews_W, linnews_b, lin2_W, lin2_b, lin3_W, lin3_b):
    src, dst = adj[0], adj[1]
    n = x.shape[0]
    e = src.shape[0]

    pad = E_PAD - e
    pad_ar = jnp.arange(pad, dtype=jnp.int32)
    src_p = jnp.concatenate([src, pad_ar % n]).reshape(-1, CHUNK)
    dst_flat = jnp.concatenate([dst, N_NODES + pad_ar % (N_ACC - N_NODES)])
    dst_p = dst_flat.reshape(-1, CHUNK)

    degp = _deg_hist(dst_flat.reshape(-1, DC))
    deg = 1.0 + (degp[0] + degp[1])
    dinv = jax.lax.rsqrt(deg)

    x_pad = jnp.pad(x, ((0, N_ACC - n), (0, 0)))
    batch_pad = jnp.concatenate(
        [batch, jnp.full((N_ACC - n,), G_GRAPHS, jnp.int32)])
    g = _matmul_scale(x_pad, conv_W1, dinv[:, None])

    partials = _edge_agg(g, src_p, dst_p)

    hp, rootsp = _conv_segmax(partials.reshape(NC, N_ACC * D), g.reshape(-1),
                              dinv, conv_b1, batch_pad)
    hp = hp.reshape(NW, G_GRAPHS, D)
    root = jnp.max(rootsp.reshape(NW, G_GRAPHS, 16)[:, :, 0], axis=0)
    news_x = x[root]

    return _heads(hp, news_x, lin2_W, lin2_b, linnews_W, linnews_b,
                  lin3_W, lin3_b)


# speedup vs baseline: 1.2968x; 1.2968x over previous
import functools

import jax
import jax.numpy as jnp
from jax import lax
from jax.experimental import pallas as pl
from jax.experimental.pallas import tpu as pltpu
from jax.experimental.pallas import tpu_sc as plsc


N_NODES = 10000
D = 128
G_GRAPHS = 128
ROW_BLK = 2048

NC = 2
NS = 16
NW = NC * NS
CHUNK = 64
CHUNKS_PER_S = 320
CPW0 = 160
CPW1 = 160
GROUP = 24
GROUP_LENS = (24, 24, 24, 24, 24, 24, 16)
E_PAD = NS * CHUNKS_PER_S * CHUNK
EDGES_PER_HIST_W = E_PAD // NW
HR = 80
N_ACC = 10240
ROWS_PER_TILE = N_ACC // NS


def _edge_agg_body(g_hbm, src_hbm, dst_hbm, out_hbm,
                   src_idx, dst_idx, buf0, buf1, buf2, acc, sem0, sem1, sem2):
    c = lax.axis_index("c")
    s = lax.axis_index("s")
    w = s * NC + c

    with jax.named_scope("agg_zero"):
        def zero_row(i, carry):
            for f in range(D // 16):
                buf0[i, pl.ds(f * 16, 16)] = jnp.zeros((16,), jnp.float32)
            return carry
        lax.fori_loop(0, CHUNK, zero_row, None)
        base = s * ROWS_PER_TILE
        for k in range(ROWS_PER_TILE // CHUNK):
            pltpu.sync_copy(buf0, acc.at[pl.ds(base + k * CHUNK, CHUNK)])
        rem = ROWS_PER_TILE % CHUNK
        if rem:
            pltpu.sync_copy(buf0.at[pl.ds(0, rem)],
                            acc.at[pl.ds(base + ROWS_PER_TILE - rem, rem)])
        plsc.subcore_barrier()

    bufs = (buf0, buf1, buf2)
    sems = (sem0, sem1, sem2)

    def start(j, b):
        pltpu.async_copy(g_hbm.at[src_idx.at[j]], bufs[b], sems[b])

    def wait(j, b):
        pltpu.make_async_copy(g_hbm.at[src_idx.at[j]], bufs[b], sems[b]).wait()

    def run_groups(first_chunk):
        done = 0
        for glen in GROUP_LENS:
            g0 = first_chunk + done
            pltpu.sync_copy(src_hbm.at[pl.ds(g0, glen)],
                            src_idx.at[pl.ds(0, glen)])
            pltpu.sync_copy(dst_hbm.at[pl.ds(g0, glen)],
                            dst_idx.at[pl.ds(0, glen)])
            for j in range(min(3, glen)):
                start(j, j % 3)
            for j in range(glen):
                b = j % 3
                wait(j, b)
                pltpu.sync_copy(bufs[b], acc.at[dst_idx.at[j]], add=True)
                if j + 3 < glen:
                    start(j + 3, b)
            done += glen

    with jax.named_scope("agg_edges"):
        base_chunk = s * CHUNKS_PER_S
        pl.when(c == 0)(lambda: run_groups(base_chunk))
        pl.when(c == 1)(lambda: run_groups(base_chunk + CPW0))
        plsc.subcore_barrier()

    with jax.named_scope("agg_writeout"):
        for k in range(ROWS_PER_TILE // CHUNK):
            b = base + k * CHUNK
            pltpu.sync_copy(acc.at[pl.ds(b, CHUNK)], buf0)
            pltpu.sync_copy(buf0, out_hbm.at[c, pl.ds(b, CHUNK)])
        if rem:
            b = base + ROWS_PER_TILE - rem
            pltpu.sync_copy(acc.at[pl.ds(b, rem)], buf0.at[pl.ds(0, rem)])
            pltpu.sync_copy(buf0.at[pl.ds(0, rem)],
                            out_hbm.at[c, pl.ds(b, rem)])


@functools.partial(
    pl.kernel,
    mesh=plsc.VectorSubcoreMesh(core_axis_name="c", subcore_axis_name="s"),
    out_type=jax.ShapeDtypeStruct((NC, N_ACC, D), jnp.float32),
    scratch_types=[
        pltpu.VMEM((GROUP, CHUNK), jnp.int32),
        pltpu.VMEM((GROUP, CHUNK), jnp.int32),
        pltpu.VMEM((CHUNK, D), jnp.float32),
        pltpu.VMEM((CHUNK, D), jnp.float32),
        pltpu.VMEM((CHUNK, D), jnp.float32),
        pltpu.VMEM_SHARED((N_ACC, D), jnp.float32),
        pltpu.SemaphoreType.DMA,
        pltpu.SemaphoreType.DMA,
        pltpu.SemaphoreType.DMA,
    ],
)
def _edge_agg(g_hbm, src_hbm, dst_hbm, out_hbm,
              src_idx, dst_idx, buf0, buf1, buf2, acc, sem0, sem1, sem2):
    _edge_agg_body(g_hbm, src_hbm, dst_hbm, out_hbm,
                   src_idx, dst_idx, buf0, buf1, buf2, acc, sem0, sem1, sem2)


DEG_HIST = NW * (E_PAD // NW // NW)
HIST_N = 10240
DC = 128
DCH = 80
DGROUP = 16


def _deg_body(dst_hbm, degp_hbm, dst_idx, ones_v, zb, wb, hist, sem):
    c = lax.axis_index("c")
    s = lax.axis_index("s")
    w = s * NC + c
    zero16 = jnp.zeros((16,), jnp.float32)

    def zrow(i, carry):
        zb[pl.ds(i * 16, 16)] = zero16
        return carry
    lax.fori_loop(0, (HIST_N // NS) // 16, zrow, None)
    pltpu.sync_copy(zb, hist.at[pl.ds(s * (HIST_N // NS), HIST_N // NS)])
    for k in range(DC // 16):
        ones_v[pl.ds(k * 16, 16)] = jnp.ones((16,), jnp.float32)
    plsc.subcore_barrier()

    for grp in range(DCH // DGROUP):
        g0 = w * DCH + grp * DGROUP
        pltpu.sync_copy(dst_hbm.at[pl.ds(g0, DGROUP)], dst_idx)
        for j in range(DGROUP):
            pltpu.async_copy(ones_v, hist.at[dst_idx.at[j]], sem, add=True)
        for j in range(DGROUP):
            pltpu.make_async_copy(ones_v, hist.at[dst_idx.at[j]], sem).wait()
    plsc.subcore_barrier()

    def wout():
        q = HIST_N // 4
        pltpu.sync_copy(hist.at[pl.ds(s * q, q)], wb)
        pltpu.sync_copy(wb, degp_hbm.at[c, pl.ds(s * q, q)])
    pl.when(s < 4)(wout)


@functools.partial(
    pl.kernel,
    mesh=plsc.VectorSubcoreMesh(core_axis_name="c", subcore_axis_name="s"),
    out_type=jax.ShapeDtypeStruct((NC, HIST_N), jnp.float32),
    scratch_types=[
        pltpu.VMEM((DGROUP, DC), jnp.int32),
        pltpu.VMEM((DC,), jnp.float32),
        pltpu.VMEM((HIST_N // NS,), jnp.float32),
        pltpu.VMEM((HIST_N // 4,), jnp.float32),
        pltpu.VMEM_SHARED((HIST_N,), jnp.float32),
        pltpu.SemaphoreType.DMA,
    ],
)
def _deg_hist(dst_hbm, degp_hbm, dst_idx, ones_v, zb, wb, hist, sem):
    _deg_body(dst_hbm, degp_hbm, dst_idx, ones_v, zb, wb, hist, sem)


SEG_ROWS_PER_W = N_ACC // NW
SEG_CHUNK = 80


def _seg_body(parts_hbm, g_hbm, dinv_hbm, b1_hbm, batch_hbm,
              segp_hbm, rootsp_hbm,
              a0b, a1b, gb, dinvv, batchv, b1v, seg, roots, sems):
    c = lax.axis_index("c")
    s = lax.axis_index("s")
    w = s * NC + c
    neg = jnp.full((16,), -jnp.inf, jnp.float32)

    def irow(i, carry):
        seg[pl.ds(i * 16, 16)] = neg
        return carry
    lax.fori_loop(0, (G_GRAPHS + 1) * D // 16, irow, None)

    def irt(i, carry):
        roots[pl.ds(i * 16, 16)] = jnp.full((16,), -1, jnp.int32)
        return carry
    lax.fori_loop(0, G_GRAPHS + 1, irt, None)

    first = w * SEG_ROWS_PER_W
    off0 = jnp.where(w == 0, 0, first - 8)
    pltpu.sync_copy(batch_hbm.at[pl.ds(off0, 16)],
                    batchv.at[0, pl.ds(0, 16)])
    cur0 = jnp.where(w == 0, jnp.int32(-1), batchv[0, pl.ds(0, 16)][7])

    pltpu.sync_copy(b1_hbm, b1v)
    b1r = [b1v[pl.ds(f * 16, 16)] for f in range(D // 16)]

    def flush(b, regs):
        for f in range(D // 16):
            seg[pl.ds(b * D + f * 16, 16)] = regs[f]

    def startch(k, ph):
        base = w * SEG_ROWS_PER_W + k * SEG_CHUNK
        pltpu.async_copy(parts_hbm.at[0, pl.ds(base * D, SEG_CHUNK * D)],
                         a0b.at[ph], sems[0])
        pltpu.async_copy(parts_hbm.at[1, pl.ds(base * D, SEG_CHUNK * D)],
                         a1b.at[ph], sems[1])
        pltpu.async_copy(g_hbm.at[pl.ds(base * D, SEG_CHUNK * D)],
                         gb.at[ph], sems[2])
        pltpu.async_copy(dinv_hbm.at[pl.ds(base, SEG_CHUNK)],
                         dinvv.at[ph, pl.ds(0, SEG_CHUNK)], sems[3])
        pltpu.async_copy(batch_hbm.at[pl.ds(base, SEG_CHUNK)],
                         batchv.at[ph, pl.ds(0, SEG_CHUNK)], sems[4])

    def waitch(k, ph):
        base = w * SEG_ROWS_PER_W + k * SEG_CHUNK
        pltpu.make_async_copy(parts_hbm.at[0, pl.ds(base * D, SEG_CHUNK * D)],
                              a0b.at[ph], sems[0]).wait()
        pltpu.make_async_copy(parts_hbm.at[1, pl.ds(base * D, SEG_CHUNK * D)],
                              a1b.at[ph], sems[1]).wait()
        pltpu.make_async_copy(g_hbm.at[pl.ds(base * D, SEG_CHUNK * D)],
                              gb.at[ph], sems[2]).wait()
        pltpu.make_async_copy(dinv_hbm.at[pl.ds(base, SEG_CHUNK)],
                              dinvv.at[ph, pl.ds(0, SEG_CHUNK)], sems[3]).wait()
        pltpu.make_async_copy(batch_hbm.at[pl.ds(base, SEG_CHUNK)],
                              batchv.at[ph, pl.ds(0, SEG_CHUNK)], sems[4]).wait()

    NCH = SEG_ROWS_PER_W // SEG_CHUNK

    startch(0, 0)
    carry = (cur0,) + tuple(neg for _ in range(D // 16))
    for k in range(NCH):
        ph = k % 2
        waitch(k, ph)
        if k + 1 < NCH:
            startch(k + 1, (k + 1) % 2)
        base = w * SEG_ROWS_PER_W + k * SEG_CHUNK

        def row_body(r, rc, ph=ph, base=base):
            cur_b = rc[0]
            regs = rc[1:]
            b = batchv[ph, pl.ds(r, 16)][0]
            dv = dinvv[ph, pl.ds(r, 16)][0]
            vals = []
            for f in range(D // 16):
                sl = pl.ds(r * D + f * 16, 16)
                v = (a0b[ph, sl] + a1b[ph, sl] + gb[ph, sl]) * dv + b1r[f]
                vals.append(jnp.maximum(v, 0.0))
            is_new = b != cur_b
            pl.when(is_new & (cur_b >= 0))(lambda: flush(cur_b, regs))
            pl.when(is_new)(lambda: roots.__setitem__(
                pl.ds(b * 16, 16), jnp.full((16,), 1, jnp.int32) * (base + r)))
            new_regs = tuple(
                jnp.where(is_new, vals[f], jnp.maximum(regs[f], vals[f]))
                for f in range(D // 16))
            return (b,) + new_regs
        carry = lax.fori_loop(0, SEG_CHUNK, row_body, carry)

    fin = carry
    pl.when(fin[0] >= 0)(lambda: flush(fin[0], fin[1:]))

    pltpu.sync_copy(seg.at[pl.ds(0, G_GRAPHS * D)], segp_hbm.at[w])
    pltpu.sync_copy(roots.at[pl.ds(0, G_GRAPHS * 16)], rootsp_hbm.at[w])


@functools.partial(
    pl.kernel,
    mesh=plsc.VectorSubcoreMesh(core_axis_name="c", subcore_axis_name="s"),
    out_type=(jax.ShapeDtypeStruct((NW, G_GRAPHS * D), jnp.float32),
              jax.ShapeDtypeStruct((NW, G_GRAPHS * 16), jnp.int32)),
    scratch_types=[
        pltpu.VMEM((2, SEG_CHUNK * D), jnp.float32),
        pltpu.VMEM((2, SEG_CHUNK * D), jnp.float32),
        pltpu.VMEM((2, SEG_CHUNK * D), jnp.float32),
        pltpu.VMEM((2, SEG_CHUNK + 16), jnp.float32),
        pltpu.VMEM((2, SEG_CHUNK + 16), jnp.int32),
        pltpu.VMEM((D,), jnp.float32),
        pltpu.VMEM(((G_GRAPHS + 1) * D,), jnp.float32),
        pltpu.VMEM(((G_GRAPHS + 1) * 16,), jnp.int32),
    ] + [pltpu.SemaphoreType.DMA] * 5,
)
def _conv_segmax(parts_hbm, g_hbm, dinv_hbm, b1_hbm, batch_hbm,
                 segp_hbm, rootsp_hbm,
                 a0b, a1b, gb, dinvv, batchv, b1v, seg, roots,
                 sm0, sm1, sm2, sm3, sm4):
    _seg_body(parts_hbm, g_hbm, dinv_hbm, b1_hbm, batch_hbm,
              segp_hbm, rootsp_hbm,
              a0b, a1b, gb, dinvv, batchv, b1v, seg, roots,
              (sm0, sm1, sm2, sm3, sm4))


def _mm_body(x_ref, w_ref, dinv_ref, g_ref):
    h = jnp.dot(x_ref[...], w_ref[...], preferred_element_type=jnp.float32)
    g_ref[...] = h * dinv_ref[...]


def _matmul_scale(x, w, dinv2d):
    n = x.shape[0]
    grid = (n // ROW_BLK,)
    return pl.pallas_call(
        _mm_body,
        grid=grid,
        in_specs=[
            pl.BlockSpec((ROW_BLK, D), lambda i: (i, 0)),
            pl.BlockSpec((D, D), lambda i: (0, 0)),
            pl.BlockSpec((ROW_BLK, 1), lambda i: (i, 0)),
        ],
        out_specs=pl.BlockSpec((ROW_BLK, D), lambda i: (i, 0)),
        out_shape=jax.ShapeDtypeStruct((n, D), jnp.float32),
    )(x, w, dinv2d)


def _heads_body(hp_ref, nx_ref, w2_ref, b2_ref, wn_ref, bn_ref,
                w3a_ref, w3b_ref, b3_ref, out_ref):
    hp = jnp.max(hp_ref[...], axis=0)
    a = jnp.maximum(
        jnp.dot(hp, w2_ref[...], preferred_element_type=jnp.float32)
        + b2_ref[...], 0.0)
    b = jnp.maximum(
        jnp.dot(nx_ref[...], wn_ref[...], preferred_element_type=jnp.float32)
        + bn_ref[...], 0.0)
    z = (jnp.dot(a, w3a_ref[...], preferred_element_type=jnp.float32)
         + jnp.dot(b, w3b_ref[...], preferred_element_type=jnp.float32)
         + b3_ref[...])
    out_ref[...] = jax.nn.sigmoid(z)


def _heads(hp, news_x, lin2_W, lin2_b, linnews_W, linnews_b, lin3_W, lin3_b):
    full = lambda s: pl.BlockSpec(s, lambda: (0,) * len(s))
    return pl.pallas_call(
        _heads_body,
        in_specs=[full((NW, G_GRAPHS, D)), full((G_GRAPHS, D)),
                  full((D, D)), full((1, D)),
                  full((D, D)), full((1, D)),
                  full((D, 1)), full((D, 1)), full((1, 1))],
        out_specs=full((G_GRAPHS, 1)),
        out_shape=jax.ShapeDtypeStruct((G_GRAPHS, 1), jnp.float32),
    )(hp, news_x, lin2_W, lin2_b.reshape(1, D), linnews_W,
      linnews_b.reshape(1, D), lin3_W[:D], lin3_W[D:], lin3_b.reshape(1, 1))


def kernel(x, adj, batch, conv_W0, conv_b0, conv_W1, conv_b1,
           linnews_W, linnews_b, lin2_W, lin2_b, lin3_W, lin3_b):
    src, dst = adj[0], adj[1]
    n = x.shape[0]
    e = src.shape[0]

    pad = E_PAD - e
    pad_ar = jnp.arange(pad, dtype=jnp.int32)
    src_p = jnp.concatenate([src, pad_ar % n]).reshape(-1, CHUNK)
    dst_flat = jnp.concatenate([dst, N_NODES + pad_ar % (N_ACC - N_NODES)])
    dst_p = dst_flat.reshape(-1, CHUNK)

    degp = _deg_hist(dst_flat.reshape(-1, DC))
    deg = 1.0 + (degp[0] + degp[1])
    dinv = jax.lax.rsqrt(deg)

    x_pad = jnp.pad(x, ((0, N_ACC - n), (0, 0)))
    batch_pad = jnp.concatenate(
        [batch, jnp.full((N_ACC - n,), G_GRAPHS, jnp.int32)])
    g = _matmul_scale(x_pad, conv_W1, dinv[:, None])

    partials = _edge_agg(g, src_p, dst_p)

    hp, rootsp = _conv_segmax(partials.reshape(NC, N_ACC * D), g.reshape(-1),
                              dinv, conv_b1, batch_pad)
    hp = hp.reshape(NW, G_GRAPHS, D)
    root = jnp.max(rootsp.reshape(NW, G_GRAPHS, 16)[:, :, 0], axis=0)
    news_x = x[root]

    return _heads(hp, news_x, lin2_W, lin2_b, linnews_W, linnews_b,
                  lin3_W, lin3_b)
